# baseline (speedup 1.0000x reference)
import functools

import jax
import jax.numpy as jnp
from jax.experimental import pallas as pl
from jax.experimental.pallas import tpu as pltpu

_EPS = 1e-5
_SUB = 4


def _cgb_kernel(x_hbm, wt_ref, gamma_ref, beta_ref, out_hbm,
                hbuf, sum8, sq8, sem_in, sem_out, *, n_rows, tile, num_tiles):
    ph = pl.program_id(0)
    t = pl.program_id(1)
    sub = tile // _SUB

    def in_copy(i, k):
        base = i * tile + k * sub
        return pltpu.make_async_copy(
            x_hbm.at[pl.ds(base, sub), :],
            hbuf.at[pl.ds(base, sub), :],
            sem_in.at[jax.lax.rem(i, 2), k])

    def out_copy(i, k):
        base = i * tile + k * sub
        return pltpu.make_async_copy(
            hbuf.at[pl.ds(base, sub), :],
            out_hbm.at[pl.ds(base, sub), :],
            sem_out.at[jax.lax.rem(i, 2), k])

    @pl.when(ph == 0)
    def _stats_phase():
        @pl.when(t == 0)
        def _init():
            sum8[...] = jnp.zeros_like(sum8)
            sq8[...] = jnp.zeros_like(sq8)
            for k in range(_SUB):
                in_copy(0, k).start()
            for k in range(_SUB):
                in_copy(1, k).start()

        @pl.when((t >= 1) & (t <= num_tiles - 2))
        def _prefetch():
            for k in range(_SUB):
                in_copy(t + 1, k).start()

        for k in range(_SUB):
            in_copy(t, k).wait()
            base = t * tile + k * sub
            xv = hbuf[pl.ds(base, sub), :]
            h = jnp.dot(xv, wt_ref[...], preferred_element_type=jnp.float32)
            h3 = h.reshape(sub // 8, 8, 128)
            sum8[...] += jnp.sum(h3, axis=0)
            sq8[...] += jnp.sum(h3 * h3, axis=0)
            hbuf[pl.ds(base, sub), :] = h

    @pl.when(ph == 1)
    def _apply_phase():
        inv_n = jnp.float32(1.0 / n_rows)
        mean = jnp.sum(sum8[...], axis=0, keepdims=True) * inv_n
        sq = jnp.sum(sq8[...], axis=0, keepdims=True) * inv_n
        var = sq - mean * mean
        scale = gamma_ref[...] * jax.lax.rsqrt(var + _EPS)
        shift = beta_ref[...] - mean * scale

        for k in range(_SUB):
            @pl.when(t > 0)
            def _drain_prev(k=k):
                out_copy(t - 1, k).wait()

            base = t * tile + k * sub
            h = hbuf[pl.ds(base, sub), :]
            hbuf[pl.ds(base, sub), :] = jnp.maximum(h * scale + shift, 0.0)
            out_copy(t, k).start()

        @pl.when(t == num_tiles - 1)
        def _drain_last():
            for k in range(_SUB):
                out_copy(t, k).wait()


@jax.jit
def kernel(p, x, o, W, gamma, beta):
    del p, o
    n, din = x.shape
    dout = W.shape[0]
    tile = 20000
    assert n % tile == 0
    num_tiles = n // tile

    wt = W.T
    gamma2 = gamma.reshape(1, dout)
    beta2 = beta.reshape(1, dout)

    out = pl.pallas_call(
        functools.partial(_cgb_kernel, n_rows=n, tile=tile,
                          num_tiles=num_tiles),
        grid=(2, num_tiles),
        in_specs=[
            pl.BlockSpec(memory_space=pltpu.MemorySpace.HBM),
            pl.BlockSpec((din, dout), lambda ph, t: (0, 0)),
            pl.BlockSpec((1, dout), lambda ph, t: (0, 0)),
            pl.BlockSpec((1, dout), lambda ph, t: (0, 0)),
        ],
        out_specs=pl.BlockSpec(memory_space=pltpu.MemorySpace.HBM),
        out_shape=jax.ShapeDtypeStruct((n, dout), jnp.float32),
        scratch_shapes=[
            pltpu.VMEM((n, dout), jnp.float32),
            pltpu.VMEM((8, dout), jnp.float32),
            pltpu.VMEM((8, dout), jnp.float32),
            pltpu.SemaphoreType.DMA((2, _SUB)),
            pltpu.SemaphoreType.DMA((2, _SUB)),
        ],
        compiler_params=pltpu.CompilerParams(
            dimension_semantics=("arbitrary", "arbitrary"),
            vmem_limit_bytes=60 * 1024 * 1024,
        ),
    )(x, wt, gamma2, beta2)
    return out


# speedup vs baseline: 1.2527x; 1.2527x over previous
import functools

import jax
import jax.numpy as jnp
from jax.experimental import pallas as pl
from jax.experimental.pallas import tpu as pltpu

_EPS = 1e-5
_SIZES = (24000, 24000, 24000, 24000, 4000)
_OFFS = (0, 24000, 48000, 72000, 96000)
_T = len(_SIZES)


def _cgb_kernel(x_hbm, wt_ref, gamma_ref, beta_ref, out_hbm,
                hbuf, sum8, sq8, sem_in, sem_out, *, n_rows):
    ph = pl.program_id(0)
    t = pl.program_id(1)

    def in_copy(i):
        return pltpu.make_async_copy(
            x_hbm.at[pl.ds(_OFFS[i], _SIZES[i]), :],
            hbuf.at[pl.ds(_OFFS[i], _SIZES[i]), :],
            sem_in.at[i])

    def out_copy(i):
        return pltpu.make_async_copy(
            hbuf.at[pl.ds(_OFFS[i], _SIZES[i]), :],
            out_hbm.at[pl.ds(_OFFS[i], _SIZES[i]), :],
            sem_out.at[i])

    @pl.when(ph == 0)
    def _stats_phase():
        @pl.when(t == 0)
        def _init():
            sum8[...] = jnp.zeros_like(sum8)
            sq8[...] = jnp.zeros_like(sq8)
            for i in range(_T):
                in_copy(i).start()

        for i in range(_T):
            @pl.when(t == i)
            def _do(i=i):
                in_copy(i).wait()
                xv = hbuf[pl.ds(_OFFS[i], _SIZES[i]), :]
                h = jnp.dot(xv, wt_ref[...],
                            preferred_element_type=jnp.float32)
                h3 = h.reshape(_SIZES[i] // 8, 8, 128)
                sum8[...] += jnp.sum(h3, axis=0)
                sq8[...] += jnp.sum(h3 * h3, axis=0)
                hbuf[pl.ds(_OFFS[i], _SIZES[i]), :] = h

    @pl.when(ph == 1)
    def _apply_phase():
        inv_n = jnp.float32(1.0 / n_rows)
        mean = jnp.sum(sum8[...], axis=0, keepdims=True) * inv_n
        sq = jnp.sum(sq8[...], axis=0, keepdims=True) * inv_n
        var = sq - mean * mean
        scale = gamma_ref[...] * jax.lax.rsqrt(var + _EPS)
        shift = beta_ref[...] - mean * scale

        for i in range(_T):
            @pl.when(t == _T - 1 - i)
            def _do(i=i):
                h = hbuf[pl.ds(_OFFS[i], _SIZES[i]), :]
                hbuf[pl.ds(_OFFS[i], _SIZES[i]), :] = jnp.maximum(
                    h * scale + shift, 0.0)
                out_copy(i).start()
                if i + 1 < _T:
                    out_copy(i + 1).wait()
                if i == 0:
                    out_copy(0).wait()


@jax.jit
def kernel(p, x, o, W, gamma, beta):
    del p, o
    n, din = x.shape
    dout = W.shape[0]
    assert n == sum(_SIZES)

    wt = W.T
    gamma2 = gamma.reshape(1, dout)
    beta2 = beta.reshape(1, dout)

    out = pl.pallas_call(
        functools.partial(_cgb_kernel, n_rows=n),
        grid=(2, _T),
        in_specs=[
            pl.BlockSpec(memory_space=pltpu.MemorySpace.HBM),
            pl.BlockSpec((din, dout), lambda ph, t: (0, 0)),
            pl.BlockSpec((1, dout), lambda ph, t: (0, 0)),
            pl.BlockSpec((1, dout), lambda ph, t: (0, 0)),
        ],
        out_specs=pl.BlockSpec(memory_space=pltpu.MemorySpace.HBM),
        out_shape=jax.ShapeDtypeStruct((n, dout), jnp.float32),
        scratch_shapes=[
            pltpu.VMEM((n, dout), jnp.float32),
            pltpu.VMEM((8, dout), jnp.float32),
            pltpu.VMEM((8, dout), jnp.float32),
            pltpu.SemaphoreType.DMA((_T,)),
            pltpu.SemaphoreType.DMA((_T,)),
        ],
        compiler_params=pltpu.CompilerParams(
            dimension_semantics=("arbitrary", "arbitrary"),
            vmem_limit_bytes=60 * 1024 * 1024,
        ),
    )(x, wt, gamma2, beta2)
    return out
